# Initial kernel scaffold; baseline (speedup 1.0000x reference)
import jax
import jax.numpy as jnp
from jax.experimental import pallas as pl


def kernel(output, target, ind, weight):
    raise NotImplementedError("write your pallas kernel here")


# speedup vs baseline: 1.6390x; 1.6390x over previous
import functools

import jax
import jax.numpy as jnp
from jax import lax
from jax.experimental import pallas as pl
from jax.experimental.pallas import tpu as pltpu
from jax.experimental.pallas import tpu_sc as plsc

_LANES = 16
_CHUNK = 128


def _sc_gather_terms(tbl16, ind_p, w_p, n_chunks, num_ch, hw):
    rows_per_plane = hw // _LANES
    batches = tbl16.shape[0] // (num_ch * rows_per_plane)
    npad = n_chunks * _CHUNK
    mesh = plsc.VectorSubcoreMesh(core_axis_name="c", subcore_axis_name="s")

    @functools.partial(
        pl.kernel,
        out_type=jax.ShapeDtypeStruct((batches, n_chunks, _CHUNK), jnp.float32),
        mesh=mesh,
        scratch_types=[
            pltpu.VMEM((n_chunks, _CHUNK), jnp.int32),
            pltpu.VMEM((n_chunks, _CHUNK), jnp.float32),
            pltpu.VMEM((n_chunks, _CHUNK), jnp.int32),
            pltpu.VMEM((num_ch * n_chunks, _CHUNK), jnp.int32),
            pltpu.VMEM((npad, _LANES), jnp.float32),
            pltpu.VMEM((npad, _LANES), jnp.float32),
            pltpu.VMEM((npad, _LANES), jnp.float32),
            pltpu.VMEM((npad, _LANES), jnp.float32),
            pltpu.VMEM((n_chunks, _CHUNK), jnp.float32),
            pltpu.SemaphoreType.DMA,
        ],
        compiler_params=pltpu.CompilerParams(
            needs_layout_passes=False, use_tc_tiling_on_sc=False),
    )
    def k(tbl_hbm, ind_hbm, w_hbm, s_hbm,
          ind_v, w_v, rem_v, idx_dma, g0, g1, g2, g3, s_v, sem):
        b = lax.axis_index("s") * 2 + lax.axis_index("c")
        pltpu.sync_copy(ind_hbm.at[b], ind_v)
        pltpu.sync_copy(w_hbm.at[b], w_v)
        gbufs = (g0, g1, g2, g3)
        for j in range(n_chunks):
            for t in range(_CHUNK // _LANES):
                sl = (j, pl.ds(t * _LANES, _LANES))
                iv = ind_v[sl]
                row = lax.shift_right_logical(iv, 4)
                rem_v[sl] = iv & (_LANES - 1)
                for c in range(num_ch):
                    base = (b * num_ch + c) * rows_per_plane
                    idx_dma[(c * n_chunks + j, pl.ds(t * _LANES, _LANES))] = (
                        row + base)
        copies = []
        for c in range(num_ch):
            for j in range(n_chunks):
                copies.append(pltpu.async_copy(
                    tbl_hbm.at[idx_dma.at[c * n_chunks + j]],
                    gbufs[c].at[pl.ds(j * _CHUNK, _CHUNK)], sem))
        for cp in copies:
            cp.wait()
        lane = lax.iota(jnp.int32, _LANES)
        for j in range(n_chunks):
            for t in range(_CHUNK // _LANES):
                sl = (j, pl.ds(t * _LANES, _LANES))
                rows16 = j * _CHUNK + t * _LANES + lane
                cols16 = rem_v[sl]
                w = w_v[sl]
                p0 = plsc.load_gather(g0, [rows16, cols16]) * w
                p1 = plsc.load_gather(g1, [rows16, cols16]) * w
                p2 = plsc.load_gather(g2, [rows16, cols16]) * w
                p3 = plsc.load_gather(g3, [rows16, cols16]) * w
                x1 = jnp.minimum(p0, p2) / (jnp.maximum(p0, p2) + 1e-6)
                x2 = jnp.minimum(p1, p3) / (jnp.maximum(p1, p3) + 1e-6)
                s_v[sl] = x1 + x2
        pltpu.sync_copy(s_v, s_hbm.at[b])

    return k(tbl16, ind_p, w_p)


def _tc_loss(s, w_p, n_valid):
    def body(s_ref, w_ref, o_ref):
        sv = s_ref[...]
        wv = w_ref[...]
        col = lax.broadcasted_iota(jnp.int32, sv.shape, 1)
        loss = jnp.where(col < n_valid, -jnp.log(sv), 0.0)
        o_ref[...] = (jnp.sum(loss) / (jnp.sum(wv) + 1e-6)).reshape(1, 1)

    return pl.pallas_call(
        body,
        out_shape=jax.ShapeDtypeStruct((1, 1), jnp.float32),
    )(s, w_p)


def kernel(output, target, ind, weight):
    del target
    b, c, h, w = output.shape
    hw = h * w
    n = ind.shape[1]
    n_chunks = -(-n // _CHUNK)
    npad = n_chunks * _CHUNK

    tbl16 = output.reshape(b * c * hw // _LANES, _LANES)
    ind_p = jnp.zeros((b, npad), jnp.int32).at[:, :n].set(ind)
    w_p = jnp.zeros((b, npad), jnp.float32).at[:, :n].set(weight)

    s = _sc_gather_terms(
        tbl16,
        ind_p.reshape(b, n_chunks, _CHUNK),
        w_p.reshape(b, n_chunks, _CHUNK),
        n_chunks, c, hw)

    loss = _tc_loss(s.reshape(b, npad), w_p, n)
    return loss[0, 0]


# speedup vs baseline: 3.0187x; 1.8418x over previous
import functools

import jax
import jax.numpy as jnp
from jax import lax
from jax.experimental import pallas as pl
from jax.experimental.pallas import tpu as pltpu
from jax.experimental.pallas import tpu_sc as plsc

_LANES = 16
_CHUNK = 128


def _sc_gather_terms(tbl16, ind_p, w_p, n_chunks, num_ch, hw):
    rows_per_plane = hw // _LANES
    batches = tbl16.shape[0] // (num_ch * rows_per_plane)
    npad = n_chunks * _CHUNK
    mesh = plsc.VectorSubcoreMesh(core_axis_name="c", subcore_axis_name="s")

    @functools.partial(
        pl.kernel,
        out_type=jax.ShapeDtypeStruct((batches, n_chunks, _CHUNK), jnp.float32),
        mesh=mesh,
        scratch_types=[
            pltpu.VMEM((n_chunks, _CHUNK), jnp.int32),
            pltpu.VMEM((n_chunks, _CHUNK), jnp.float32),
            pltpu.VMEM((n_chunks, _CHUNK), jnp.int32),
            pltpu.VMEM((num_ch * n_chunks, _CHUNK), jnp.int32),
            pltpu.VMEM((npad, _LANES), jnp.float32),
            pltpu.VMEM((npad, _LANES), jnp.float32),
            pltpu.VMEM((npad, _LANES), jnp.float32),
            pltpu.VMEM((npad, _LANES), jnp.float32),
            pltpu.VMEM((n_chunks, _CHUNK), jnp.float32),
            pltpu.SemaphoreType.DMA,
        ],
        compiler_params=pltpu.CompilerParams(
            needs_layout_passes=False, use_tc_tiling_on_sc=False),
    )
    def k(tbl_hbm, ind_hbm, w_hbm, s_hbm,
          ind_v, w_v, rem_v, idx_dma, g0, g1, g2, g3, s_v, sem):
        b = lax.axis_index("s") * 2 + lax.axis_index("c")
        pltpu.sync_copy(ind_hbm.at[b], ind_v)
        pltpu.sync_copy(w_hbm.at[b], w_v)
        gbufs = (g0, g1, g2, g3)
        for j in range(n_chunks):
            for t in range(_CHUNK // _LANES):
                sl = (j, pl.ds(t * _LANES, _LANES))
                iv = ind_v[sl]
                addr = (
                    lax.shift_right_logical(iv, 11) * 2048
                    + (lax.shift_right_logical(iv, 7) & 1) * 1024
                    + (lax.shift_right_logical(iv, 8) & 7) * 128
                    + (iv & 127))
                row = lax.shift_right_logical(addr, 4)
                rem_v[sl] = iv & (_LANES - 1)
                for c in range(num_ch):
                    base = (b * num_ch + c) * rows_per_plane
                    idx_dma[(c * n_chunks + j, pl.ds(t * _LANES, _LANES))] = (
                        row + base)
        copies = []
        for c in range(num_ch):
            for j in range(n_chunks):
                copies.append(pltpu.async_copy(
                    tbl_hbm.at[idx_dma.at[c * n_chunks + j]],
                    gbufs[c].at[pl.ds(j * _CHUNK, _CHUNK)], sem))
        for cp in copies:
            cp.wait()
        lane = lax.iota(jnp.int32, _LANES)
        for j in range(n_chunks):
            for t in range(_CHUNK // _LANES):
                sl = (j, pl.ds(t * _LANES, _LANES))
                rows16 = j * _CHUNK + t * _LANES + lane
                cols16 = rem_v[sl]
                w = w_v[sl]
                p0 = plsc.load_gather(g0, [rows16, cols16]) * w
                p1 = plsc.load_gather(g1, [rows16, cols16]) * w
                p2 = plsc.load_gather(g2, [rows16, cols16]) * w
                p3 = plsc.load_gather(g3, [rows16, cols16]) * w
                x1 = jnp.minimum(p0, p2) / (jnp.maximum(p0, p2) + 1e-6)
                x2 = jnp.minimum(p1, p3) / (jnp.maximum(p1, p3) + 1e-6)
                s_v[sl] = x1 + x2
        pltpu.sync_copy(s_v, s_hbm.at[b])

    return k(tbl16, ind_p, w_p)


def _tc_loss(s, w_p, n_valid):
    def body(s_ref, w_ref, o_ref):
        sv = s_ref[...]
        wv = w_ref[...]
        col = lax.broadcasted_iota(jnp.int32, sv.shape, 1)
        loss = jnp.where(col < n_valid, -jnp.log(sv), 0.0)
        o_ref[...] = (jnp.sum(loss) / (jnp.sum(wv) + 1e-6)).reshape(1, 1)

    return pl.pallas_call(
        body,
        out_shape=jax.ShapeDtypeStruct((1, 1), jnp.float32),
    )(s, w_p)


def kernel(output, target, ind, weight):
    del target
    b, c, h, w = output.shape
    hw = h * w
    n = ind.shape[1]
    n_chunks = -(-n // _CHUNK)
    npad = n_chunks * _CHUNK

    tbl16 = (
        output.reshape(b, c, h // 8, 8, w // 128, 128)
        .transpose(0, 1, 2, 4, 3, 5)
        .reshape(b * c * hw // _LANES, _LANES))
    ind_p = jnp.zeros((b, npad), jnp.int32).at[:, :n].set(ind)
    w_p = jnp.zeros((b, npad), jnp.float32).at[:, :n].set(weight)

    s = _sc_gather_terms(
        tbl16,
        ind_p.reshape(b, n_chunks, _CHUNK),
        w_p.reshape(b, n_chunks, _CHUNK),
        n_chunks, c, hw)

    loss = _tc_loss(s.reshape(b, npad), w_p, n)
    return loss[0, 0]
